# baseline (speedup 1.0000x reference)
import jax
import jax.numpy as jnp
from jax.experimental import pallas as pl

N = 1024
D = 768
F = 3072
RADIUS = 2.0
BM = 256

_BF = jnp.bfloat16


def _qkv_body(x_ref, g1_ref, b1_ref, Wqkv_ref, bqkv_ref, q_ref, k_ref, v_ref):
    x = x_ref[...]
    mu = jnp.mean(x, axis=-1, keepdims=True)
    var = jnp.mean((x - mu) ** 2, axis=-1, keepdims=True)
    h = (x - mu) / jnp.sqrt(var + 1e-5) * g1_ref[...] + b1_ref[...]
    qkv = jnp.dot(h.astype(_BF), Wqkv_ref[...],
                  preferred_element_type=jnp.float32) + bqkv_ref[...]
    qkv = qkv.astype(_BF)
    q_ref[...] = qkv[:, :D]
    k_ref[...] = qkv[:, D:2 * D]
    v_ref[...] = qkv[:, 2 * D:]


def _attn_ffn_body(x_ref, q_ref, k_ref, v_ref, gxc_ref, gyc_ref, gxr_ref,
                   gyr_ref, Wo_ref, bo_ref, g2_ref, b2_ref, W1_ref, bf1_ref,
                   W2_ref, bf2_ref, o_ref):
    i = pl.program_id(0)
    q = q_ref[...]
    k = k_ref[...]
    v = v_ref[...]

    scores = jax.lax.dot_general(
        q, k, (((1,), (1,)), ((), ())),
        preferred_element_type=jnp.float32) * (1.0 / (D ** 0.5))

    dx = jnp.abs(gxc_ref[...] - gxr_ref[...])
    dy = jnp.abs(gyc_ref[...] - gyr_ref[...])
    rows = i * BM + jax.lax.broadcasted_iota(jnp.int32, (BM, N), 0)
    cols = jax.lax.broadcasted_iota(jnp.int32, (BM, N), 1)
    mask = (dx <= RADIUS) & (dy <= RADIUS) & (rows != cols)

    neg = jnp.finfo(jnp.float32).min
    s = jnp.where(mask, scores, neg)
    m = jnp.max(s, axis=-1, keepdims=True)
    e = jnp.exp(s - m) * mask.astype(jnp.float32)
    denom = jnp.sum(e, axis=-1, keepdims=True)
    attn = e / jnp.maximum(denom, 1e-30)
    out = jnp.dot(attn.astype(_BF), v, preferred_element_type=jnp.float32)

    has_nbr = jnp.any(mask, axis=-1, keepdims=True)
    v_blk = v_ref[pl.ds(i * BM, BM), :].astype(jnp.float32)
    out = jnp.where(has_nbr, out, v_blk)

    x_new = x_ref[...] + jnp.dot(out.astype(_BF), Wo_ref[...],
                                 preferred_element_type=jnp.float32) + bo_ref[...]

    mu = jnp.mean(x_new, axis=-1, keepdims=True)
    var = jnp.mean((x_new - mu) ** 2, axis=-1, keepdims=True)
    h2 = (x_new - mu) / jnp.sqrt(var + 1e-5) * g2_ref[...] + b2_ref[...]

    t = jnp.dot(h2.astype(_BF), W1_ref[...],
                preferred_element_type=jnp.float32) + bf1_ref[...]
    g = 0.5 * t * (1.0 + jax.lax.erf(t * (2.0 ** -0.5)))
    f = jnp.dot(g.astype(_BF), W2_ref[...],
                preferred_element_type=jnp.float32) + bf2_ref[...]
    o_ref[...] = x_new + f


def kernel(x, grid, Wq, bq, Wk, bk, Wv, bv, Wo, bo, g1, b1n, g2, b2n, W1, bf1, W2, bf2):
    Wqkv = jnp.concatenate([Wq, Wk, Wv], axis=1).astype(_BF)
    bqkv = jnp.concatenate([bq, bk, bv]).reshape(1, 3 * D)
    g1r = g1.reshape(1, D)
    b1r = b1n.reshape(1, D)

    nblk = N // BM
    q, k, v = pl.pallas_call(
        _qkv_body,
        grid=(nblk,),
        in_specs=[
            pl.BlockSpec((BM, D), lambda i: (i, 0)),
            pl.BlockSpec((1, D), lambda i: (0, 0)),
            pl.BlockSpec((1, D), lambda i: (0, 0)),
            pl.BlockSpec((D, 3 * D), lambda i: (0, 0)),
            pl.BlockSpec((1, 3 * D), lambda i: (0, 0)),
        ],
        out_specs=[
            pl.BlockSpec((BM, D), lambda i: (i, 0)),
            pl.BlockSpec((BM, D), lambda i: (i, 0)),
            pl.BlockSpec((BM, D), lambda i: (i, 0)),
        ],
        out_shape=[jax.ShapeDtypeStruct((N, D), _BF)] * 3,
    )(x, g1r, b1r, Wqkv, bqkv)

    gf = grid.astype(jnp.float32)
    gxc = gf[:, 0:1]
    gyc = gf[:, 1:2]
    gxr = gf[:, 0].reshape(1, N)
    gyr = gf[:, 1].reshape(1, N)

    out = pl.pallas_call(
        _attn_ffn_body,
        grid=(nblk,),
        in_specs=[
            pl.BlockSpec((BM, D), lambda i: (i, 0)),
            pl.BlockSpec((BM, D), lambda i: (i, 0)),
            pl.BlockSpec((N, D), lambda i: (0, 0)),
            pl.BlockSpec((N, D), lambda i: (0, 0)),
            pl.BlockSpec((BM, 1), lambda i: (i, 0)),
            pl.BlockSpec((BM, 1), lambda i: (i, 0)),
            pl.BlockSpec((1, N), lambda i: (0, 0)),
            pl.BlockSpec((1, N), lambda i: (0, 0)),
            pl.BlockSpec((D, D), lambda i: (0, 0)),
            pl.BlockSpec((1, D), lambda i: (0, 0)),
            pl.BlockSpec((1, D), lambda i: (0, 0)),
            pl.BlockSpec((1, D), lambda i: (0, 0)),
            pl.BlockSpec((D, F), lambda i: (0, 0)),
            pl.BlockSpec((1, F), lambda i: (0, 0)),
            pl.BlockSpec((F, D), lambda i: (0, 0)),
            pl.BlockSpec((1, D), lambda i: (0, 0)),
        ],
        out_specs=pl.BlockSpec((BM, D), lambda i: (i, 0)),
        out_shape=jax.ShapeDtypeStruct((N, D), jnp.float32),
    )(x, q, k, v, gxc, gyc, gxr, gyr,
      Wo.astype(_BF), bo.reshape(1, D), g2.reshape(1, D), b2n.reshape(1, D),
      W1.astype(_BF), bf1.reshape(1, F), W2.astype(_BF), bf2.reshape(1, D))
    return out


# speedup vs baseline: 1.2627x; 1.2627x over previous
import jax
import jax.numpy as jnp
from jax.experimental import pallas as pl
from jax.experimental.pallas import tpu as pltpu

N = 1024
D = 768
F = 3072
RADIUS = 2.0
BM = 256
NBLK = N // BM


def _body(x_ref, gxc_ref, gxr_ref, gyc_ref, gyr_ref,
          Wq_ref, Wk_ref, Wv_ref, bqkv_ref,
          g1_ref, b1_ref, Wo_ref, bo_ref, g2_ref, b2_ref,
          W1_ref, bf1_ref, W2_ref, bf2_ref,
          o_ref, q_s, k_s, v_s):
    i = pl.program_id(0)

    @pl.when(i == 0)
    def _qkv():
        x = x_ref[...]
        mu = jnp.mean(x, axis=-1, keepdims=True)
        var = jnp.mean((x - mu) ** 2, axis=-1, keepdims=True)
        h = (x - mu) / jnp.sqrt(var + 1e-5) * g1_ref[...] + b1_ref[...]
        b = bqkv_ref[...]
        q_s[...] = jnp.dot(h, Wq_ref[...], preferred_element_type=jnp.float32) + b[0:1, :]
        k_s[...] = jnp.dot(h, Wk_ref[...], preferred_element_type=jnp.float32) + b[1:2, :]
        v_s[...] = jnp.dot(h, Wv_ref[...], preferred_element_type=jnp.float32) + b[2:3, :]

    @pl.when(i > 0)
    def _attn_ffn():
        j = i - 1
        q = q_s[pl.ds(j * BM, BM), :]
        k = k_s[...]
        v = v_s[...]

        scores = jax.lax.dot_general(
            q, k, (((1,), (1,)), ((), ())),
            preferred_element_type=jnp.float32) * (1.0 / (D ** 0.5))

        gxc = gxc_ref[pl.ds(j * BM, BM), :]
        gyc = gyc_ref[pl.ds(j * BM, BM), :]
        dx = jnp.abs(gxc - gxr_ref[...])
        dy = jnp.abs(gyc - gyr_ref[...])
        rows = j * BM + jax.lax.broadcasted_iota(jnp.int32, (BM, N), 0)
        cols = jax.lax.broadcasted_iota(jnp.int32, (BM, N), 1)
        mask = (dx <= RADIUS) & (dy <= RADIUS) & (rows != cols)

        neg = jnp.finfo(jnp.float32).min
        s = jnp.where(mask, scores, neg)
        m = jnp.max(s, axis=-1, keepdims=True)
        e = jnp.exp(s - m) * mask.astype(jnp.float32)
        denom = jnp.sum(e, axis=-1, keepdims=True)
        attn = e / jnp.maximum(denom, 1e-30)
        out = jnp.dot(attn, v, preferred_element_type=jnp.float32)

        has_nbr = jnp.any(mask, axis=-1, keepdims=True)
        v_blk = v_s[pl.ds(j * BM, BM), :]
        out = jnp.where(has_nbr, out, v_blk)

        x_new = x_ref[pl.ds(j * BM, BM), :] + jnp.dot(
            out, Wo_ref[...], preferred_element_type=jnp.float32) + bo_ref[...]

        mu = jnp.mean(x_new, axis=-1, keepdims=True)
        var = jnp.mean((x_new - mu) ** 2, axis=-1, keepdims=True)
        h2 = (x_new - mu) / jnp.sqrt(var + 1e-5) * g2_ref[...] + b2_ref[...]

        t = jnp.dot(h2, W1_ref[...], preferred_element_type=jnp.float32) + bf1_ref[...]
        g = 0.5 * t * (1.0 + jax.lax.erf(t * (2.0 ** -0.5)))
        f = jnp.dot(g, W2_ref[...], preferred_element_type=jnp.float32) + bf2_ref[...]
        o_ref[...] = x_new + f


def kernel(x, grid, Wq, bq, Wk, bk, Wv, bv, Wo, bo, g1, b1n, g2, b2n, W1, bf1, W2, bf2):
    gf = grid.astype(jnp.float32)
    gxc = gf[:, 0:1]
    gyc = gf[:, 1:2]
    gxr = gf[:, 0].reshape(1, N)
    gyr = gf[:, 1].reshape(1, N)
    bqkv = jnp.stack([bq, bk, bv])

    const = lambda i: (0, 0)
    full = lambda shape: pl.BlockSpec(shape, const)

    out = pl.pallas_call(
        _body,
        grid=(NBLK + 1,),
        in_specs=[
            full((N, D)),
            full((N, 1)),
            full((1, N)),
            full((N, 1)),
            full((1, N)),
            full((D, D)),
            full((D, D)),
            full((D, D)),
            full((3, D)),
            full((1, D)),
            full((1, D)),
            full((D, D)),
            full((1, D)),
            full((1, D)),
            full((1, D)),
            full((D, F)),
            full((1, F)),
            full((F, D)),
            full((1, D)),
        ],
        out_specs=pl.BlockSpec((BM, D), lambda i: (jax.lax.max(i - 1, 0), 0)),
        out_shape=jax.ShapeDtypeStruct((N, D), jnp.float32),
        scratch_shapes=[
            pltpu.VMEM((N, D), jnp.float32),
            pltpu.VMEM((N, D), jnp.float32),
            pltpu.VMEM((N, D), jnp.float32),
        ],
    )(x, gxc, gxr, gyc, gyr, Wq, Wk, Wv, bqkv,
      g1.reshape(1, D), b1n.reshape(1, D), Wo, bo.reshape(1, D),
      g2.reshape(1, D), b2n.reshape(1, D),
      W1, bf1.reshape(1, F), W2, bf2.reshape(1, D))
    return out
